# baseline (speedup 1.0000x reference)
import functools
import jax
import jax.numpy as jnp
from jax import lax
from jax.experimental import pallas as pl
from jax.experimental.pallas import tpu as pltpu
from jax.experimental.pallas import tpu_sc as plsc

NC, NS = 2, 16
NW = NC * NS
D = 64
CHUNK = 128


NBUF = 8
LAG = 4


@functools.partial(jax.jit, static_argnames=("n_rows",))
def _gather_rows(idx2d, table, n_rows):
    n_chunks_total = idx2d.shape[0]
    n_chunks = n_chunks_total // NW
    rows_per_w = n_chunks * CHUNK

    mesh = plsc.VectorSubcoreMesh(
        core_axis_name="c", subcore_axis_name="s",
        num_cores=NC, num_subcores=NS)

    @functools.partial(
        pl.kernel,
        out_type=jax.ShapeDtypeStruct((n_rows, D), jnp.float32),
        mesh=mesh,
        scratch_types=[
            pltpu.VMEM((n_chunks, CHUNK), jnp.int32),
            pltpu.VMEM((NBUF, CHUNK, D), jnp.float32),
        ] + [pltpu.SemaphoreType.DMA] * (2 * NBUF),
        compiler_params=pltpu.CompilerParams(use_tc_tiling_on_sc=False),
    )
    def k(idx_hbm, table_hbm, out_hbm, idx_v, rows_v, *sems):
        gsems = sems[:NBUF]
        ssems = sems[NBUF:]
        wid = lax.axis_index("s") * NC + lax.axis_index("c")
        pltpu.sync_copy(idx_hbm.at[pl.ds(wid * n_chunks, n_chunks)], idx_v)
        row_base = wid * rows_per_w

        def start_gather(j, b):
            pltpu.async_copy(table_hbm.at[idx_v.at[j]], rows_v.at[b], gsems[b])

        def wait_gather(b):
            pltpu.make_async_copy(
                table_hbm.at[idx_v.at[0]], rows_v.at[b], gsems[b]).wait()

        def start_store(j, b):
            pltpu.async_copy(
                rows_v.at[b], out_hbm.at[pl.ds(row_base + j * CHUNK, CHUNK)],
                ssems[b])

        def wait_store(j, b):
            pltpu.make_async_copy(
                rows_v.at[b], out_hbm.at[pl.ds(row_base + j * CHUNK, CHUNK)],
                ssems[b]).wait()

        for b in range(LAG):
            start_gather(b, b)
        for j in range(LAG):
            start_gather(j + LAG, j + LAG)
            wait_gather(j)
            start_store(j, j)

        @pl.loop(LAG, n_chunks - LAG, step=NBUF)
        def body(g):
            for i in range(NBUF):
                j = g + i
                mj = (LAG + i) % NBUF
                mg = (2 * LAG + i) % NBUF
                wait_store(j - (NBUF - LAG), mg)
                start_gather(j + LAG, mg)
                wait_gather(mj)
                start_store(j, mj)

        for j in range(n_chunks - LAG, n_chunks):
            b = j % NBUF
            wait_gather(b)
            start_store(j, b)
        for j in range(n_chunks - NBUF, n_chunks):
            wait_store(j, j % NBUF)

    return k(idx2d, table)


def kernel(token_ids, embeddings):
    b, s = token_ids.shape
    n_rows = b * s
    idx2d = token_ids.astype(jnp.int32).reshape(n_rows // CHUNK, CHUNK)
    out = _gather_rows(idx2d, embeddings, n_rows)
    return out.reshape(b, s, D)


# speedup vs baseline: 1.0026x; 1.0026x over previous
import functools
import jax
import jax.numpy as jnp
from jax import lax
from jax.experimental import pallas as pl
from jax.experimental.pallas import tpu as pltpu
from jax.experimental.pallas import tpu_sc as plsc

NC, NS = 2, 16
NW = NC * NS
D = 64
CHUNK = 128
NBUF = 8
LAG = 4


@jax.jit
def _gather_rows(idx_t, table):
    S, B = idx_t.shape
    b_per_w = B // NW
    groups = b_per_w // CHUNK
    n_chunks = S * groups

    mesh = plsc.VectorSubcoreMesh(
        core_axis_name="c", subcore_axis_name="s",
        num_cores=NC, num_subcores=NS)

    @functools.partial(
        pl.kernel,
        out_type=jax.ShapeDtypeStruct((B, S, D), jnp.float32),
        mesh=mesh,
        scratch_types=[
            pltpu.VMEM((S, b_per_w), jnp.int32),
            pltpu.VMEM((NBUF, CHUNK, D), jnp.float32),
        ] + [pltpu.SemaphoreType.DMA] * (2 * NBUF),
        compiler_params=pltpu.CompilerParams(use_tc_tiling_on_sc=False),
    )
    def k(idx_hbm, table_hbm, out_hbm, idx_v, rows_v, *sems):
        gsems = sems[:NBUF]
        ssems = sems[NBUF:]
        wid = lax.axis_index("s") * NC + lax.axis_index("c")
        b_base = wid * b_per_w
        pltpu.sync_copy(idx_hbm.at[:, pl.ds(b_base, b_per_w)], idx_v)

        def chunk_pos(q):
            return q // groups, (q % groups) * CHUNK

        def start_gather(q, b):
            s, boff = chunk_pos(q)
            pltpu.async_copy(
                table_hbm.at[idx_v.at[s, pl.ds(boff, CHUNK)]],
                rows_v.at[b], gsems[b])

        def wait_gather(b):
            pltpu.make_async_copy(
                table_hbm.at[idx_v.at[0, pl.ds(0, CHUNK)]],
                rows_v.at[b], gsems[b]).wait()

        def out_slice(q):
            s, boff = chunk_pos(q)
            return out_hbm.at[pl.ds(b_base + boff, CHUNK), s]

        def start_store(q, b):
            pltpu.async_copy(rows_v.at[b], out_slice(q), ssems[b])

        def wait_store(q, b):
            pltpu.make_async_copy(rows_v.at[b], out_slice(q), ssems[b]).wait()

        for b in range(LAG):
            start_gather(b, b)
        for q in range(LAG):
            start_gather(q + LAG, q + LAG)
            wait_gather(q)
            start_store(q, q)

        @pl.loop(LAG, n_chunks - LAG, step=NBUF)
        def body(g):
            for i in range(NBUF):
                q = g + i
                mq = (LAG + i) % NBUF
                mg = (2 * LAG + i) % NBUF
                wait_store(q - (NBUF - LAG), mg)
                start_gather(q + LAG, mg)
                wait_gather(mq)
                start_store(q, mq)

        for j in range(n_chunks - LAG, n_chunks):
            b = j % NBUF
            wait_gather(b)
            start_store(j, b)
        for j in range(n_chunks - NBUF, n_chunks):
            wait_store(j, j % NBUF)

    return k(idx_t, table)


def kernel(token_ids, embeddings):
    idx_t = token_ids.astype(jnp.int32).T
    return _gather_rows(idx_t, embeddings)
